# Initial kernel scaffold; baseline (speedup 1.0000x reference)
import jax
import jax.numpy as jnp
from jax.experimental import pallas as pl


def kernel(prediction, target):
    raise NotImplementedError("write your pallas kernel here")


# speedup vs baseline: 12.7657x; 12.7657x over previous
import jax
import jax.numpy as jnp
from jax.experimental import pallas as pl

TRIM = 0.2
B = 8
N = 512 * 512
KOFF = int((1.0 - TRIM) * N)
ROWS_2D = 2048
LANES = 128


def _row_kernel(pred_ref, tgt_ref, out_ref):
    pred = pred_ref[0]
    tgt = tgt_ref[0]
    mask = tgt > 0
    res = jnp.where(mask, jnp.abs(pred - tgt), 0.0)
    nmask = jnp.sum(mask.astype(jnp.int32))

    k = N - nmask + KOFF
    k = jnp.minimum(k, N - 1)

    bits = jax.lax.bitcast_convert_type(res, jnp.int32)

    def body(i, carry):
        prefix, rank = carry
        shift = 31 - i
        shifted = jax.lax.shift_right_logical(bits, shift)
        c0 = jnp.sum((shifted == 2 * prefix).astype(jnp.int32))
        go_right = (rank >= c0).astype(jnp.int32)
        prefix = 2 * prefix + go_right
        rank = rank - go_right * c0
        return prefix, rank

    t_bits, _ = jax.lax.fori_loop(0, 32, body, (jnp.int32(0), k))

    kept = jnp.where(bits <= t_bits, res, 0.0)
    s = jnp.sum(kept)
    norm = jnp.maximum(2 * nmask, 1).astype(jnp.float32)
    loss = jnp.where(nmask > 0, s / norm, 0.0)
    out_ref[0, 0, :] = jnp.full((LANES,), loss, dtype=jnp.float32)


def kernel(prediction, target):
    pred = prediction.reshape(B, ROWS_2D, LANES)
    tgt = target.reshape(B, ROWS_2D, LANES)
    losses = pl.pallas_call(
        _row_kernel,
        grid=(B,),
        in_specs=[
            pl.BlockSpec((1, ROWS_2D, LANES), lambda i: (i, 0, 0)),
            pl.BlockSpec((1, ROWS_2D, LANES), lambda i: (i, 0, 0)),
        ],
        out_specs=pl.BlockSpec((1, 1, LANES), lambda i: (i, 0, 0)),
        out_shape=jax.ShapeDtypeStruct((B, 1, LANES), jnp.float32),
    )(pred, tgt)
    return jnp.mean(losses[:, 0, 0])


# speedup vs baseline: 16.2989x; 1.2768x over previous
import jax
import jax.numpy as jnp
from jax.experimental import pallas as pl

TRIM = 0.2
B = 8
N = 512 * 512
KOFF = int((1.0 - TRIM) * N)
ROWS_2D = 2048
LANES = 128


def _row_kernel(pred_ref, tgt_ref, out_ref):
    pred = pred_ref[0]
    tgt = tgt_ref[0]
    mask = tgt > 0
    res = jnp.where(mask, jnp.abs(pred - tgt), 0.0)
    nmask = jnp.sum(mask.astype(jnp.int32))

    k = N - nmask + KOFF
    k = jnp.minimum(k, N - 1)

    bits = jax.lax.bitcast_convert_type(res, jnp.int32)

    def body(i, carry):
        prefix, rank = carry
        b = 30 - i
        mid = jax.lax.shift_left(2 * prefix + 1, b)
        cnt_lt_mid = jnp.sum((bits < mid).astype(jnp.int32))
        c0 = cnt_lt_mid - (k - rank)
        go_right = (rank >= c0).astype(jnp.int32)
        prefix = 2 * prefix + go_right
        rank = rank - go_right * c0
        return prefix, rank

    t_bits, _ = jax.lax.fori_loop(0, 31, body, (jnp.int32(0), k))

    kept = jnp.where(bits <= t_bits, res, 0.0)
    s = jnp.sum(kept)
    norm = jnp.maximum(2 * nmask, 1).astype(jnp.float32)
    loss = jnp.where(nmask > 0, s / norm, 0.0)
    out_ref[0, 0, :] = jnp.full((LANES,), loss, dtype=jnp.float32)


def kernel(prediction, target):
    pred = prediction.reshape(B, ROWS_2D, LANES)
    tgt = target.reshape(B, ROWS_2D, LANES)
    losses = pl.pallas_call(
        _row_kernel,
        grid=(B,),
        in_specs=[
            pl.BlockSpec((1, ROWS_2D, LANES), lambda i: (i, 0, 0)),
            pl.BlockSpec((1, ROWS_2D, LANES), lambda i: (i, 0, 0)),
        ],
        out_specs=pl.BlockSpec((1, 1, LANES), lambda i: (i, 0, 0)),
        out_shape=jax.ShapeDtypeStruct((B, 1, LANES), jnp.float32),
    )(pred, tgt)
    return jnp.mean(losses[:, 0, 0])
